# baseline (speedup 1.0000x reference)
import functools

import jax
import jax.numpy as jnp
from jax import lax
from jax.experimental import pallas as pl
from jax.experimental.pallas import tpu as pltpu
from jax.experimental.pallas import tpu_sc as plsc

N_NODES = 10000
C = 128
B = 200
NPG = 50
T_STEPS = 2 * NPG
M_EDGES = 320000

SC_TILES = 16
EDGES_PAD = 327680
EDGES_PER_TILE = EDGES_PAD // SC_TILES
CHUNK = 4096
CHUNK_ROWS = CHUNK // 128
N_CHUNKS = EDGES_PER_TILE // CHUNK
TABLE_LIVE = NPG * B * NPG
TABLE_SIZE = 512000
ZBUF = 8000
OUT_BLK = 4000


def _sc_hist_body(edges_ref, out_ref, table, ebuf, idxbuf, ones, zbuf, cbuf):
    c = lax.axis_index("c")
    s = lax.axis_index("s")
    lanes = lax.iota(jnp.int32, 16)

    def fill_z(i, carry):
        zbuf[pl.ds(i * 16, 16)] = jnp.zeros((16,), jnp.float32)
        return carry

    lax.fori_loop(0, ZBUF // 16, fill_z, 0)

    def fill_o(i, carry):
        ones[pl.ds(i * 16, 16)] = jnp.full((16,), 1.0, jnp.float32)
        return carry

    lax.fori_loop(0, 128 // 16, fill_o, 0)

    stripe = TABLE_SIZE // SC_TILES
    for k in range(stripe // ZBUF):
        pltpu.sync_copy(zbuf, table.at[pl.ds(s * stripe + k * ZBUF, ZBUF)])
    plsc.subcore_barrier()

    def chunk_step(k, carry):
        cb = s * EDGES_PER_TILE + k * CHUNK
        for r in range(4):
            pltpu.sync_copy(edges_ref.at[r, pl.ds(cb, CHUNK)], ebuf.at[r])
        for j in range(CHUNK_ROWS):
            def lane_step(l, carry2):
                off = j * 128 + l * 16
                src = ebuf[0, pl.ds(off, 16)]
                dst = ebuf[1, pl.ds(off, 16)]
                dis = ebuf[2, pl.ds(off, 16)]
                g = ebuf[3, pl.ds(off, 16)]
                idx1 = dst * N_NODES + src * B + g
                idx0 = (NPG - 1 - src) * N_NODES + dst * B + g
                idx = jnp.where(c == 0, idx1, idx0)
                dead = TABLE_LIVE + ((off + cb) & 4095) + lanes
                idx = jnp.where(dis == 1, idx, dead)
                idxbuf[j, pl.ds(l * 16, 16)] = idx
                return carry2

            lax.fori_loop(0, 128 // 16, lane_step, 0)
        def scat(j, carry2):
            pltpu.sync_copy(ones, table.at[idxbuf.at[j]], add=True)
            return carry2

        lax.fori_loop(0, CHUNK_ROWS, scat, 0)
        return carry

    lax.fori_loop(0, N_CHUNKS, chunk_step, 0)
    plsc.subcore_barrier()

    n_blocks = TABLE_LIVE // OUT_BLK

    def out_step(i, carry):
        q = s + i * SC_TILES

        @pl.when(q < n_blocks)
        def _():
            pltpu.sync_copy(table.at[pl.ds(q * OUT_BLK, OUT_BLK)], cbuf)
            pltpu.sync_copy(cbuf, out_ref.at[pl.ds(c * TABLE_LIVE + q * OUT_BLK, OUT_BLK)])

        return carry

    lax.fori_loop(0, (n_blocks + SC_TILES - 1) // SC_TILES, out_step, 0)


def _sc_histogram(edges):
    et = jnp.pad(edges.T.astype(jnp.int32), ((0, 0), (0, EDGES_PAD - M_EDGES)))
    mesh = plsc.VectorSubcoreMesh(core_axis_name="c", subcore_axis_name="s")
    hist = pl.kernel(
        _sc_hist_body,
        out_type=jax.ShapeDtypeStruct((2 * TABLE_LIVE,), jnp.float32),
        mesh=mesh,
        scratch_types=[
            pltpu.VMEM_SHARED((TABLE_SIZE,), jnp.float32),
            pltpu.VMEM((4, CHUNK), jnp.int32),
            pltpu.VMEM((CHUNK_ROWS, 128), jnp.int32),
            pltpu.VMEM((128,), jnp.float32),
            pltpu.VMEM((ZBUF,), jnp.float32),
            pltpu.VMEM((OUT_BLK,), jnp.float32),
        ],
    )(et)
    return hist.reshape(T_STEPS, NPG, B)


def _gnn_tc_kernel(n_ref, x_ref, cnt_ref, w1_ref, b1_ref, w2_ref, b2_ref,
                   wih_ref, whh_ref, bih_ref, bhh_ref, out_ref, msg_ref):
    out_ref[...] = x_ref[...]

    w1 = w1_ref[...]
    b1 = b1_ref[...]
    w2 = w2_ref[...]
    b2 = b2_ref[...]
    wih = wih_ref[...]
    whh = whh_ref[...]
    bih = bih_ref[...]
    bhh = bhh_ref[...]
    n_local_t = n_ref[0]

    def _mlp(h):
        h = jnp.maximum(jnp.dot(w1, h, preferred_element_type=jnp.float32) + b1, 0.0)
        return jnp.maximum(jnp.dot(w2, h, preferred_element_type=jnp.float32) + b2, 0.0)

    for i in range(NPG):
        msg_ref[i] = _mlp(x_ref[i])

    def _combine(cnt):
        m = cnt[0:1, :] * msg_ref[0]
        for s in range(1, NPG):
            m = m + cnt[s:s + 1, :] * msg_ref[s]
        return m

    def step(t, m):
        node = jnp.where(t < NPG, t, (T_STEPS - 1) - t)
        tn = jnp.minimum(t + 1, T_STEPS - 1)
        m_pre = _combine(cnt_ref[pl.ds(tn, 1)][0])
        msg_old = msg_ref[pl.ds(node, 1)][0]
        hprev = out_ref[pl.ds(node, 1)][0]
        gi = jnp.dot(wih, m, preferred_element_type=jnp.float32) + bih
        gh = jnp.dot(whh, hprev, preferred_element_type=jnp.float32) + bhh
        r = jax.nn.sigmoid(gi[0:C] + gh[0:C])
        z = jax.nn.sigmoid(gi[C:2 * C] + gh[C:2 * C])
        n = jnp.tanh(gi[2 * C:3 * C] + r * gh[2 * C:3 * C])
        hnew = (1.0 - z) * n + z * hprev
        hsel = jnp.where(node < n_local_t, hnew, hprev)
        msg_new = _mlp(hsel)
        out_ref[pl.ds(node, 1)] = hsel[None]
        msg_ref[pl.ds(node, 1)] = msg_new[None]
        cdel = cnt_ref[pl.ds(tn, 1), pl.ds(node, 1)][0]
        return m_pre + cdel * (msg_new - msg_old)

    m0 = _combine(cnt_ref[pl.ds(0, 1)][0])
    jax.lax.fori_loop(0, T_STEPS, step, m0)


def kernel(x, edges, node_graph_ind, mlp_w1, mlp_b1, mlp_w2, mlp_b2,
           gru_w_ih, gru_w_hh, gru_b_ih, gru_b_hh):
    del node_graph_ind
    cnt = _sc_histogram(edges)
    n_local_t = (jnp.max(edges[:, 1]) + 1).astype(jnp.int32).reshape(1)
    x_cm = x.reshape(B, NPG, C).transpose(1, 2, 0)

    vmem = pl.BlockSpec(memory_space=pltpu.VMEM)
    out = pl.pallas_call(
        _gnn_tc_kernel,
        out_shape=jax.ShapeDtypeStruct((NPG, C, B), jnp.float32),
        in_specs=[pl.BlockSpec(memory_space=pltpu.SMEM)] + [vmem] * 10,
        out_specs=vmem,
        scratch_shapes=[pltpu.VMEM((NPG, C, B), jnp.float32)],
    )(n_local_t, x_cm, cnt,
      mlp_w1, jnp.broadcast_to(mlp_b1[:, None], (C, B)),
      mlp_w2, jnp.broadcast_to(mlp_b2[:, None], (C, B)),
      gru_w_ih, gru_w_hh,
      jnp.broadcast_to(gru_b_ih[:, None], (3 * C, B)),
      jnp.broadcast_to(gru_b_hh[:, None], (3 * C, B)))

    return out.transpose(2, 0, 1).reshape(N_NODES, C)


# speedup vs baseline: 1.0369x; 1.0369x over previous
import functools

import jax
import jax.numpy as jnp
from jax import lax
from jax.experimental import pallas as pl
from jax.experimental.pallas import tpu as pltpu
from jax.experimental.pallas import tpu_sc as plsc

N_NODES = 10000
C = 128
B = 200
NPG = 50
T_STEPS = 2 * NPG
M_EDGES = 320000

SC_TILES = 16
EDGES_PAD = 327680
EDGES_PER_TILE = EDGES_PAD // SC_TILES
CHUNK = 4096
CHUNK_ROWS = CHUNK // 128
N_CHUNKS = EDGES_PER_TILE // CHUNK
TABLE_LIVE = NPG * B * NPG
TABLE_SIZE = 512000
ZBUF = 8000
OUT_BLK = 4000


def _sc_hist_body(edges_ref, out_ref, table, ebuf, idxbuf, ones, zbuf, cbuf):
    c = lax.axis_index("c")
    s = lax.axis_index("s")
    lanes = lax.iota(jnp.int32, 16)

    def fill_z(i, carry):
        zbuf[pl.ds(i * 16, 16)] = jnp.zeros((16,), jnp.float32)
        return carry

    lax.fori_loop(0, ZBUF // 16, fill_z, 0)

    def fill_o(i, carry):
        ones[pl.ds(i * 16, 16)] = jnp.full((16,), 1.0, jnp.float32)
        return carry

    lax.fori_loop(0, 128 // 16, fill_o, 0)

    stripe = TABLE_SIZE // SC_TILES
    for k in range(stripe // ZBUF):
        pltpu.sync_copy(zbuf, table.at[pl.ds(s * stripe + k * ZBUF, ZBUF)])
    plsc.subcore_barrier()

    def chunk_step(k, carry):
        cb = s * EDGES_PER_TILE + k * CHUNK
        for r in range(4):
            pltpu.sync_copy(edges_ref.at[r, pl.ds(cb, CHUNK)], ebuf.at[r])
        for j in range(CHUNK_ROWS):
            def lane_step(l, carry2):
                off = j * 128 + l * 16
                src = ebuf[0, pl.ds(off, 16)]
                dst = ebuf[1, pl.ds(off, 16)]
                dis = ebuf[2, pl.ds(off, 16)]
                g = ebuf[3, pl.ds(off, 16)]
                idx1 = dst * N_NODES + src * B + g
                idx0 = (NPG - 1 - src) * N_NODES + dst * B + g
                idx = jnp.where(c == 0, idx1, idx0)
                dead = TABLE_LIVE + ((off + cb) & 4095) + lanes
                idx = jnp.where(dis == 1, idx, dead)
                idxbuf[j, pl.ds(l * 16, 16)] = idx
                return carry2

            lax.fori_loop(0, 128 // 16, lane_step, 0)
        def scat(j, carry2):
            pltpu.sync_copy(ones, table.at[idxbuf.at[j]], add=True)
            return carry2

        lax.fori_loop(0, CHUNK_ROWS, scat, 0)
        return carry

    lax.fori_loop(0, N_CHUNKS, chunk_step, 0)
    plsc.subcore_barrier()

    n_blocks = TABLE_LIVE // OUT_BLK

    def out_step(i, carry):
        q = s + i * SC_TILES

        @pl.when(q < n_blocks)
        def _():
            pltpu.sync_copy(table.at[pl.ds(q * OUT_BLK, OUT_BLK)], cbuf)
            pltpu.sync_copy(cbuf, out_ref.at[pl.ds(c * TABLE_LIVE + q * OUT_BLK, OUT_BLK)])

        return carry

    lax.fori_loop(0, (n_blocks + SC_TILES - 1) // SC_TILES, out_step, 0)


def _sc_histogram(edges):
    et = jnp.pad(edges.T.astype(jnp.int32), ((0, 0), (0, EDGES_PAD - M_EDGES)))
    mesh = plsc.VectorSubcoreMesh(core_axis_name="c", subcore_axis_name="s")
    hist = pl.kernel(
        _sc_hist_body,
        out_type=jax.ShapeDtypeStruct((2 * TABLE_LIVE,), jnp.float32),
        mesh=mesh,
        scratch_types=[
            pltpu.VMEM_SHARED((TABLE_SIZE,), jnp.float32),
            pltpu.VMEM((4, CHUNK), jnp.int32),
            pltpu.VMEM((CHUNK_ROWS, 128), jnp.int32),
            pltpu.VMEM((128,), jnp.float32),
            pltpu.VMEM((ZBUF,), jnp.float32),
            pltpu.VMEM((OUT_BLK,), jnp.float32),
        ],
    )(et)
    return hist.reshape(T_STEPS, NPG, B)


def _gnn_tc_kernel(n_ref, x_ref, cnt_ref, w1_ref, b1_ref, w2_ref, b2_ref,
                   wg_ref, bg_ref, out_ref, msg_ref):
    out_ref[...] = x_ref[...]

    w1 = w1_ref[...]
    b1 = b1_ref[...]
    w2 = w2_ref[...]
    b2 = b2_ref[...]
    wg = wg_ref[...]
    bg = bg_ref[...]
    n_local_t = n_ref[0]

    def _mlp(h):
        h = jnp.maximum(jnp.dot(w1, h, preferred_element_type=jnp.float32) + b1, 0.0)
        return jnp.maximum(jnp.dot(w2, h, preferred_element_type=jnp.float32) + b2, 0.0)

    for i in range(NPG):
        msg_ref[i] = _mlp(x_ref[i])

    def step(t, carry):
        node = jnp.where(t < NPG, t, (T_STEPS - 1) - t)
        cnt = cnt_ref[pl.ds(t, 1)][0]
        m = cnt[0:1, :] * msg_ref[0]
        for s in range(1, NPG):
            m = m + cnt[s:s + 1, :] * msg_ref[s]
        hprev = out_ref[pl.ds(node, 1)][0]
        mh = jnp.concatenate([m, hprev], axis=0)
        g = jnp.dot(wg, mh, preferred_element_type=jnp.float32) + bg
        r = jax.nn.sigmoid(g[0:C])
        z = jax.nn.sigmoid(g[C:2 * C])
        n = jnp.tanh(g[2 * C:3 * C] + r * g[3 * C:4 * C])
        hnew = (1.0 - z) * n + z * hprev
        hsel = jnp.where(node < n_local_t, hnew, hprev)
        out_ref[pl.ds(node, 1)] = hsel[None]
        msg_ref[pl.ds(node, 1)] = _mlp(hsel)[None]
        return carry

    jax.lax.fori_loop(0, T_STEPS, step, 0)


def kernel(x, edges, node_graph_ind, mlp_w1, mlp_b1, mlp_w2, mlp_b2,
           gru_w_ih, gru_w_hh, gru_b_ih, gru_b_hh):
    del node_graph_ind
    cnt = _sc_histogram(edges)
    n_local_t = (jnp.max(edges[:, 1]) + 1).astype(jnp.int32).reshape(1)
    x_cm = x.reshape(B, NPG, C).transpose(1, 2, 0)

    zeros_cc = jnp.zeros((C, C), jnp.float32)
    wg = jnp.concatenate([
        jnp.concatenate([gru_w_ih[0:2 * C], gru_w_hh[0:2 * C]], axis=1),
        jnp.concatenate([gru_w_ih[2 * C:3 * C], zeros_cc], axis=1),
        jnp.concatenate([zeros_cc, gru_w_hh[2 * C:3 * C]], axis=1),
    ], axis=0)
    bgv = jnp.concatenate([gru_b_ih[0:2 * C] + gru_b_hh[0:2 * C],
                           gru_b_ih[2 * C:3 * C], gru_b_hh[2 * C:3 * C]])

    vmem = pl.BlockSpec(memory_space=pltpu.VMEM)
    out = pl.pallas_call(
        _gnn_tc_kernel,
        out_shape=jax.ShapeDtypeStruct((NPG, C, B), jnp.float32),
        in_specs=[pl.BlockSpec(memory_space=pltpu.SMEM)] + [vmem] * 8,
        out_specs=vmem,
        scratch_shapes=[pltpu.VMEM((NPG, C, B), jnp.float32)],
    )(n_local_t, x_cm, cnt,
      mlp_w1, jnp.broadcast_to(mlp_b1[:, None], (C, B)),
      mlp_w2, jnp.broadcast_to(mlp_b2[:, None], (C, B)),
      wg, jnp.broadcast_to(bgv[:, None], (4 * C, B)))

    return out.transpose(2, 0, 1).reshape(N_NODES, C)
